# Initial kernel scaffold; baseline (speedup 1.0000x reference)
import jax
import jax.numpy as jnp
from jax.experimental import pallas as pl


def kernel(x, edge_index, W1l, b1, W1r, W2l, b2, W2r):
    raise NotImplementedError("write your pallas kernel here")


# speedup vs baseline: 8.5995x; 8.5995x over previous
import functools

import jax
import jax.numpy as jnp
from jax import lax
from jax.experimental import pallas as pl
from jax.experimental.pallas import tpu as pltpu
from jax.experimental.pallas import tpu_sc as plsc

_NS = 16
_NC = 2
_B = 128


def _ceil_to(a, m):
    return (a + m - 1) // m * m


def _proj1_body(x_ref, w_ref, b_ref, p_ref, r_ref):
    y = jnp.dot(x_ref[...], w_ref[...], preferred_element_type=jnp.float32)
    h = p_ref.shape[1]
    p_ref[...] = y[:, :h]
    r_ref[...] = y[:, h:] + b_ref[...]


def _mid_body(acc_a, acc_b, c_a, c_b, r1, w2, b2, p2, r2, cnt_out):
    cnt = c_a[...] + c_b[...]
    cnt_out[...] = cnt
    mean = (acc_a[...] + acc_b[...]) / jnp.maximum(cnt, 1.0)
    hid = jnp.maximum(mean + r1[...], 0.0)
    y = jnp.dot(hid, w2[...], preferred_element_type=jnp.float32)
    o = p2.shape[1]
    p2[...] = y[:, :o]
    r2[...] = y[:, o:] + b2[...]


def _final_body(acc_a, acc_b, cnt, r2, out):
    out[...] = (acc_a[...] + acc_b[...]) / jnp.maximum(cnt[...], 1.0) + r2[...]


def _make_segsum(n_pad, width, chunks_per_tile, with_cnt):
    rows_per_sub = n_pad // _NS
    mesh = plsc.VectorSubcoreMesh(core_axis_name="c", subcore_axis_name="s")

    out_type = [jax.ShapeDtypeStruct((_NC, n_pad, width), jnp.float32)]
    scratch = [
        pltpu.VMEM_SHARED((n_pad, width), jnp.float32),
        pltpu.VMEM((1, _B), jnp.int32),
        pltpu.VMEM((1, _B), jnp.int32),
        pltpu.VMEM((_B, width), jnp.float32),
        pltpu.SemaphoreType.DMA,
    ]
    if with_cnt:
        out_type.append(jax.ShapeDtypeStruct((_NC, n_pad, 8), jnp.float32))
        scratch += [
            pltpu.VMEM_SHARED((n_pad, 8), jnp.float32),
            pltpu.VMEM((_B, 8), jnp.float32),
        ]

    def body(*refs):
        if with_cnt:
            (p_hbm, src_hbm, dst_hbm, z2, z1, ones_h, acc_out, cnt_out,
             acc_sh, src_v, dst_v, rows_v, sem, cnt_sh, ones_v) = refs
        else:
            (p_hbm, src_hbm, dst_hbm, z2, acc_out,
             acc_sh, src_v, dst_v, rows_v, sem) = refs

        cid = lax.axis_index("c")
        sid = lax.axis_index("s")
        sl = pl.ds(sid * rows_per_sub, rows_per_sub)

        pltpu.sync_copy(z2.at[sl], acc_sh.at[sl])
        if with_cnt:
            pltpu.sync_copy(z1.at[sl], cnt_sh.at[sl])
            pltpu.sync_copy(ones_h, ones_v)
        plsc.subcore_barrier()

        base = (cid * _NS + sid) * chunks_per_tile

        def step(i, carry):
            row = base + i
            pltpu.sync_copy(src_hbm.at[row], src_v)
            pltpu.sync_copy(dst_hbm.at[row], dst_v)
            pltpu.async_copy(p_hbm.at[src_v.at[0]], rows_v, sem).wait()
            pltpu.sync_copy(rows_v, acc_sh.at[dst_v.at[0]], add=True)
            if with_cnt:
                pltpu.sync_copy(ones_v, cnt_sh.at[dst_v.at[0]], add=True)
            return carry

        lax.fori_loop(0, chunks_per_tile, step, 0)
        plsc.subcore_barrier()

        pltpu.sync_copy(acc_sh.at[sl], acc_out.at[cid, sl])
        if with_cnt:
            pltpu.sync_copy(cnt_sh.at[sl], cnt_out.at[cid, sl])

    return pl.kernel(
        body, out_type=out_type, mesh=mesh, scratch_types=scratch,
        compiler_params=pltpu.CompilerParams(use_tc_tiling_on_sc=False))


def kernel(x, edge_index, W1l, b1, W1r, W2l, b2, W2r):
    n, d = x.shape
    h = W1l.shape[0]
    o = W2l.shape[0]
    e = edge_index.shape[1]

    n_chunks = _ceil_to(-(-e // _B), _NC * _NS)
    ep = n_chunks * _B
    cpt = n_chunks // (_NC * _NS)
    n_pad = _ceil_to(n + 1, _NS * 8)

    src = edge_index[0]
    dst = edge_index[1]
    pad = ep - e
    srcp = jnp.concatenate([src, jnp.zeros((pad,), jnp.int32)]).reshape(n_chunks, 1, _B)
    dstp = jnp.concatenate([dst, jnp.full((pad,), n, jnp.int32)]).reshape(n_chunks, 1, _B)

    z_h = jnp.zeros((n_pad, h), jnp.float32)
    z_o = jnp.zeros((n_pad, o), jnp.float32)
    z_1 = jnp.zeros((n_pad, 8), jnp.float32)

    w1 = jnp.concatenate([W1l, W1r], axis=0).T
    p1, r1 = pl.pallas_call(
        _proj1_body,
        out_shape=[jax.ShapeDtypeStruct((n, h), jnp.float32),
                   jax.ShapeDtypeStruct((n, h), jnp.float32)],
    )(x, w1, b1.reshape(1, h))

    ones_e = jnp.ones((_B, 8), jnp.float32)
    acc1, cnt1 = _make_segsum(n_pad, h, cpt, True)(p1, srcp, dstp, z_h, z_1, ones_e)

    w2 = jnp.concatenate([W2l, W2r], axis=0).T
    p2, r2, cnt = pl.pallas_call(
        _mid_body,
        out_shape=[jax.ShapeDtypeStruct((n, o), jnp.float32),
                   jax.ShapeDtypeStruct((n, o), jnp.float32),
                   jax.ShapeDtypeStruct((n, 1), jnp.float32)],
    )(acc1[0, :n], acc1[1, :n], cnt1[0, :n, :1], cnt1[1, :n, :1],
      r1, w2, b2.reshape(1, o))

    (acc2,) = _make_segsum(n_pad, o, cpt, False)(p2, srcp, dstp, z_o)

    out = pl.pallas_call(
        _final_body,
        out_shape=jax.ShapeDtypeStruct((n, o), jnp.float32),
    )(acc2[0, :n], acc2[1, :n], cnt, r2)
    return out


# speedup vs baseline: 15.1800x; 1.7652x over previous
import functools

import jax
import jax.numpy as jnp
from jax import lax
from jax.experimental import pallas as pl
from jax.experimental.pallas import tpu as pltpu
from jax.experimental.pallas import tpu_sc as plsc

_NS = 16
_NC = 2
_B = 128


def _ceil_to(a, m):
    return (a + m - 1) // m * m


def _proj1_body(x_ref, w_ref, b_ref, p_ref, r_ref):
    y = jnp.dot(x_ref[...], w_ref[...], preferred_element_type=jnp.float32)
    h = p_ref.shape[1]
    p_ref[...] = y[:, :h]
    r_ref[...] = y[:, h:] + b_ref[...]


def _mid_body(acc_a, acc_b, c_a, c_b, r1, w2, b2, p2, r2, cnt_out):
    cnt = c_a[...] + c_b[...]
    cnt_out[...] = cnt
    mean = (acc_a[...] + acc_b[...]) / jnp.maximum(cnt, 1.0)
    hid = jnp.maximum(mean + r1[...], 0.0)
    y = jnp.dot(hid, w2[...], preferred_element_type=jnp.float32)
    o = p2.shape[1]
    p2[...] = y[:, :o]
    r2[...] = y[:, o:] + b2[...]


def _final_body(acc_a, acc_b, cnt, r2, out):
    out[...] = (acc_a[...] + acc_b[...]) / jnp.maximum(cnt[...], 1.0) + r2[...]


_SUP = 8


def _make_segsum(n_pad, width, chunks_per_tile, with_cnt):
    rows_per_sub = n_pad // _NS
    n_sup = chunks_per_tile // _SUP
    mesh = plsc.VectorSubcoreMesh(core_axis_name="c", subcore_axis_name="s")

    out_type = [jax.ShapeDtypeStruct((_NC, n_pad, width), jnp.float32)]
    scratch = [
        pltpu.VMEM_SHARED((n_pad, width), jnp.float32),
        [pltpu.VMEM((_SUP, _B), jnp.int32) for _ in range(2)],
        [pltpu.VMEM((_SUP, _B), jnp.int32) for _ in range(2)],
        [[pltpu.VMEM((_B, width), jnp.float32) for _ in range(_SUP)]
         for _ in range(2)],
        [[pltpu.SemaphoreType.DMA for _ in range(_SUP)]
         for _ in range(2)],
    ]
    if with_cnt:
        out_type.append(jax.ShapeDtypeStruct((_NC, n_pad, 8), jnp.float32))
        scratch += [
            pltpu.VMEM_SHARED((n_pad, 8), jnp.float32),
            pltpu.VMEM((_B, 8), jnp.float32),
        ]

    def body(*refs):
        if with_cnt:
            (p_hbm, src_hbm, dst_hbm, z2, z1, ones_h, acc_out, cnt_out,
             acc_sh, src_v, dst_v, rows, gsem, cnt_sh, ones_v) = refs
        else:
            (p_hbm, src_hbm, dst_hbm, z2, acc_out,
             acc_sh, src_v, dst_v, rows, gsem) = refs

        cid = lax.axis_index("c")
        sid = lax.axis_index("s")
        sl = pl.ds(sid * rows_per_sub, rows_per_sub)

        pltpu.sync_copy(z2.at[sl], acc_sh.at[sl])
        if with_cnt:
            pltpu.sync_copy(z1.at[sl], cnt_sh.at[sl])
            pltpu.sync_copy(ones_h, ones_v)
        plsc.subcore_barrier()

        sup_base = (cid * _NS + sid) * n_sup
        n_pairs = n_sup // 2

        def load_and_fire(s, par):
            pltpu.sync_copy(src_hbm.at[s], src_v[par])
            pltpu.sync_copy(dst_hbm.at[s], dst_v[par])
            for j in range(_SUP):
                pltpu.async_copy(p_hbm.at[src_v[par].at[j]],
                                 rows[par][j], gsem[par][j])

        def drain_and_scatter(par):
            for j in range(_SUP):
                pltpu.make_async_copy(p_hbm.at[src_v[par].at[j]],
                                      rows[par][j], gsem[par][j]).wait()
                pltpu.sync_copy(rows[par][j],
                                acc_sh.at[dst_v[par].at[j]], add=True)
                if with_cnt:
                    pltpu.sync_copy(ones_v,
                                    cnt_sh.at[dst_v[par].at[j]], add=True)

        load_and_fire(sup_base, 0)

        def pair(i, carry):
            load_and_fire(sup_base + 2 * i + 1, 1)
            drain_and_scatter(0)

            @pl.when(i < n_pairs - 1)
            def _():
                load_and_fire(sup_base + 2 * i + 2, 0)
            drain_and_scatter(1)
            return carry

        lax.fori_loop(0, n_pairs, pair, 0)
        plsc.subcore_barrier()

        pltpu.sync_copy(acc_sh.at[sl], acc_out.at[cid, sl])
        if with_cnt:
            pltpu.sync_copy(cnt_sh.at[sl], cnt_out.at[cid, sl])

    return pl.kernel(
        body, out_type=out_type, mesh=mesh, scratch_types=scratch,
        compiler_params=pltpu.CompilerParams(use_tc_tiling_on_sc=False))


def kernel(x, edge_index, W1l, b1, W1r, W2l, b2, W2r):
    n, d = x.shape
    h = W1l.shape[0]
    o = W2l.shape[0]
    e = edge_index.shape[1]

    n_chunks = _ceil_to(-(-e // _B), _NC * _NS * _SUP * 2)
    ep = n_chunks * _B
    cpt = n_chunks // (_NC * _NS)
    n_sups = n_chunks // _SUP
    n_pad = _ceil_to(n + 1, _NS * 8)

    src = edge_index[0]
    dst = edge_index[1]
    pad = ep - e
    srcp = jnp.concatenate([src, jnp.zeros((pad,), jnp.int32)]).reshape(n_sups, _SUP, _B)
    dstp = jnp.concatenate([dst, jnp.full((pad,), n, jnp.int32)]).reshape(n_sups, _SUP, _B)

    z_h = jnp.zeros((n_pad, h), jnp.float32)
    z_o = jnp.zeros((n_pad, o), jnp.float32)
    z_1 = jnp.zeros((n_pad, 8), jnp.float32)

    w1 = jnp.concatenate([W1l, W1r], axis=0).T
    p1, r1 = pl.pallas_call(
        _proj1_body,
        out_shape=[jax.ShapeDtypeStruct((n, h), jnp.float32),
                   jax.ShapeDtypeStruct((n, h), jnp.float32)],
    )(x, w1, b1.reshape(1, h))

    ones_e = jnp.ones((_B, 8), jnp.float32)
    acc1, cnt1 = _make_segsum(n_pad, h, cpt, True)(p1, srcp, dstp, z_h, z_1, ones_e)

    w2 = jnp.concatenate([W2l, W2r], axis=0).T
    p2, r2, cnt = pl.pallas_call(
        _mid_body,
        out_shape=[jax.ShapeDtypeStruct((n, o), jnp.float32),
                   jax.ShapeDtypeStruct((n, o), jnp.float32),
                   jax.ShapeDtypeStruct((n, 1), jnp.float32)],
    )(acc1[0, :n], acc1[1, :n], cnt1[0, :n, :1], cnt1[1, :n, :1],
      r1, w2, b2.reshape(1, o))

    (acc2,) = _make_segsum(n_pad, o, cpt, False)(p2, srcp, dstp, z_o)

    out = pl.pallas_call(
        _final_body,
        out_shape=jax.ShapeDtypeStruct((n, o), jnp.float32),
    )(acc2[0, :n], acc2[1, :n], cnt, r2)
    return out
